# baseline (speedup 1.0000x reference)
import jax
import jax.numpy as jnp
from jax import lax
from jax.experimental import pallas as pl
from jax.experimental.pallas import tpu as pltpu
from jax.experimental.pallas import tpu_sc as plsc

E = 0.1
CENTER = 1.0 - E
B, S, L = 16, 2048, 512
N = B * S
NC, NS = 2, 16
NW = NC * NS
TPW = N // NW
HALF = TPW // 2
ROWS = 8192
NBLK = N // ROWS


def _sc_body(labels_hbm, logits_hbm, dot_hbm, wsum_hbm,
             labs_v, xb_v, dot_v, wsum_v):
    cid = lax.axis_index("c")
    sid = lax.axis_index("s")
    wid = sid * NC + cid
    base = wid * TPW
    @pl.when(wid == 0)
    def _():
        pltpu.sync_copy(labels_hbm.at[pl.ds(0, TPW + 8)],
                        labs_v.at[pl.ds(8, TPW + 8)])

    @pl.when(wid == NW - 1)
    def _():
        pltpu.sync_copy(labels_hbm.at[pl.ds(N - TPW - 8, TPW + 8)],
                        labs_v.at[pl.ds(0, TPW + 8)])

    @pl.when((wid > 0) & (wid < NW - 1))
    def _():
        pltpu.sync_copy(labels_hbm.at[pl.ds(base - 8, TPW + 16)], labs_v)

    cbase = (wid % (S // TPW)) * TPW

    for h in range(TPW // HALF):
        pltpu.sync_copy(
            logits_hbm.at[pl.ds(base + h * HALF, HALF), pl.ds(0, 128)], xb_v)

        def group(g, carry):
            t0 = h * HALF + g * 16
            lane = lax.iota(jnp.int32, 16)
            tloc = t0 + lane
            c = cbase + tloc
            rs = []
            for j in range(-2, 3):
                rj = labs_v[pl.ds(t0 + 8 + j, 16)]
                vj = ((c + j) >= 0) & ((c + j) < S)
                rs.append(jnp.where(vj, rj, -1))
            dot = jnp.zeros(16, jnp.float32)
            ws = jnp.zeros(16, jnp.float32)
            for j in range(-2, 3):
                rj = rs[j + 2]
                bnd = (rj >= 0) & (rj < 16) & ((rj & 1) == 1)
                keep = bnd
                for jp in range(j + 1, 3):
                    keep = keep & (rs[jp + 2] != rj)
                if j == 0:
                    val = jnp.full(16, CENTER, jnp.float32)
                else:
                    cp = c + j
                    val = jnp.full(16, E / 4, jnp.float32)
                    val = jnp.where((cp == 1) | (cp == S - 2), E / 3, val)
                    val = jnp.where((cp == 0) | (cp == S - 1), E / 2, val)
                w = jnp.where(keep, val, 0.0)
                gj = plsc.load_gather(
                    xb_v, [g * 16 + lane, jnp.where(keep, rj, 0)])
                dot = dot + w * gj
                ws = ws + w
            r0 = rs[2]
            bnd0 = (r0 < 16) & ((r0 & 1) == 1)
            ws = ws + jnp.where(bnd0, 0.0, 1.0)
            dot_v[pl.ds(t0, 16)] = dot
            wsum_v[pl.ds(t0, 16)] = ws
            return carry

        lax.fori_loop(0, HALF // 16, group, 0)

    pltpu.sync_copy(dot_v, dot_hbm.at[pl.ds(base, TPW)])
    pltpu.sync_copy(wsum_v, wsum_hbm.at[pl.ds(base, TPW)])


def _sc_sparse_part(logits, label_ids):
    mesh = plsc.VectorSubcoreMesh(core_axis_name="c", subcore_axis_name="s",
                                  num_cores=NC, num_subcores=NS)
    k = pl.kernel(
        _sc_body,
        out_type=(jax.ShapeDtypeStruct((N,), jnp.float32),
                  jax.ShapeDtypeStruct((N,), jnp.float32)),
        mesh=mesh,
        scratch_types=[
            pltpu.VMEM((TPW + 16,), jnp.int32),
            pltpu.VMEM((HALF, 128), jnp.float32),
            pltpu.VMEM((TPW,), jnp.float32),
            pltpu.VMEM((TPW,), jnp.float32),
        ],
        compiler_params=pltpu.CompilerParams(needs_layout_passes=False),
    )
    return k(label_ids, logits)


def _tc_lse_body(x_ref, lab_ref, lse_ref, own_ref):
    i = pl.program_id(0)
    x = x_ref[0]
    r0 = lab_ref[...][:, None]
    bnd0 = (r0 < 16) & (r0 % 2 == 1)
    iota_l = jax.lax.broadcasted_iota(jnp.int32, (ROWS, L), 1)
    g0 = jnp.sum(jnp.where(iota_l == r0, x, 0.0), axis=1, keepdims=True)
    m = jnp.max(x, axis=1, keepdims=True)
    lse = m + jnp.log(jnp.sum(jnp.exp(x - m), axis=1, keepdims=True))
    lse_ref[...] = lse[:, 0]

    @pl.when(i == 0)
    def _():
        own_ref[0, 0, 0] = 0.0
    own_ref[0, 0, 0] += jnp.sum(jnp.where(bnd0, 0.0, g0))


def _tc_combine_body(lse_ref, dot_ref, wsum_ref, own_ref, out_ref):
    tot = jnp.sum(wsum_ref[...] * lse_ref[...] - dot_ref[...])
    out_ref[0, 0, 0] = (tot - own_ref[0, 0, 0]) / N


@jax.jit
def kernel(logits, label_ids):
    dot, wsum = _sc_sparse_part(logits, label_ids)
    lse, own = pl.pallas_call(
        _tc_lse_body,
        grid=(NBLK,),
        in_specs=[
            pl.BlockSpec((1, ROWS, L), lambda i: (i, 0, 0)),
            pl.BlockSpec((ROWS,), lambda i: (i,)),
        ],
        out_specs=[
            pl.BlockSpec((ROWS,), lambda i: (i,)),
            pl.BlockSpec((1, 1, 1), lambda i: (0, 0, 0),
                         memory_space=pltpu.SMEM),
        ],
        out_shape=[
            jax.ShapeDtypeStruct((N,), jnp.float32),
            jax.ShapeDtypeStruct((1, 1, 1), jnp.float32),
        ],
    )(logits.reshape(NBLK, ROWS, L), label_ids)
    combined = pl.pallas_call(
        _tc_combine_body,
        grid=(1,),
        in_specs=[
            pl.BlockSpec((N,), lambda i: (0,)),
            pl.BlockSpec((N,), lambda i: (0,)),
            pl.BlockSpec((N,), lambda i: (0,)),
            pl.BlockSpec(memory_space=pltpu.SMEM),
        ],
        out_specs=pl.BlockSpec((1, 1, 1), lambda i: (0, 0, 0),
                               memory_space=pltpu.SMEM),
        out_shape=jax.ShapeDtypeStruct((1, 1, 1), jnp.float32),
    )(lse, dot, wsum, own)
    return combined[0, 0, 0]


# speedup vs baseline: 1.0370x; 1.0370x over previous
import jax
import jax.numpy as jnp
from jax import lax
from jax.experimental import pallas as pl
from jax.experimental.pallas import tpu as pltpu
from jax.experimental.pallas import tpu_sc as plsc

E = 0.1
CENTER = 1.0 - E
B, S, L = 16, 2048, 512
N = B * S
NC, NS = 2, 16
NW = NC * NS
TPW = N // NW
HALF = TPW // 2
ROWS = 4096
NBLK = N // ROWS


def _sc_body(labels_hbm, logits_hbm, dot_hbm, wsum_hbm,
             labs_v, xb_v, dot_v, wsum_v):
    cid = lax.axis_index("c")
    sid = lax.axis_index("s")
    wid = sid * NC + cid
    base = wid * TPW
    @pl.when(wid == 0)
    def _():
        pltpu.sync_copy(labels_hbm.at[pl.ds(0, TPW + 8)],
                        labs_v.at[pl.ds(8, TPW + 8)])

    @pl.when(wid == NW - 1)
    def _():
        pltpu.sync_copy(labels_hbm.at[pl.ds(N - TPW - 8, TPW + 8)],
                        labs_v.at[pl.ds(0, TPW + 8)])

    @pl.when((wid > 0) & (wid < NW - 1))
    def _():
        pltpu.sync_copy(labels_hbm.at[pl.ds(base - 8, TPW + 16)], labs_v)

    cbase = (wid % (S // TPW)) * TPW

    for h in range(TPW // HALF):
        pltpu.sync_copy(
            logits_hbm.at[pl.ds(base + h * HALF, HALF), pl.ds(0, 128)], xb_v)

        def group(g, carry):
            t0 = h * HALF + g * 16
            lane = lax.iota(jnp.int32, 16)
            tloc = t0 + lane
            c = cbase + tloc
            rs = []
            for j in range(-2, 3):
                rj = labs_v[pl.ds(t0 + 8 + j, 16)]
                vj = ((c + j) >= 0) & ((c + j) < S)
                rs.append(jnp.where(vj, rj, -1))
            dot = jnp.zeros(16, jnp.float32)
            ws = jnp.zeros(16, jnp.float32)
            for j in range(-2, 3):
                rj = rs[j + 2]
                bnd = (rj >= 0) & (rj < 16) & ((rj & 1) == 1)
                keep = bnd
                for jp in range(j + 1, 3):
                    keep = keep & (rs[jp + 2] != rj)
                if j == 0:
                    val = jnp.full(16, CENTER, jnp.float32)
                else:
                    cp = c + j
                    val = jnp.full(16, E / 4, jnp.float32)
                    val = jnp.where((cp == 1) | (cp == S - 2), E / 3, val)
                    val = jnp.where((cp == 0) | (cp == S - 1), E / 2, val)
                w = jnp.where(keep, val, 0.0)
                gj = plsc.load_gather(
                    xb_v, [g * 16 + lane, jnp.where(keep, rj, 0)])
                dot = dot + w * gj
                ws = ws + w
            r0 = rs[2]
            bnd0 = (r0 < 16) & ((r0 & 1) == 1)
            ws = ws + jnp.where(bnd0, 0.0, 1.0)
            dot_v[pl.ds(t0, 16)] = dot
            wsum_v[pl.ds(t0, 16)] = ws
            return carry

        lax.fori_loop(0, HALF // 16, group, 0)

    pltpu.sync_copy(dot_v, dot_hbm.at[pl.ds(base, TPW)])
    pltpu.sync_copy(wsum_v, wsum_hbm.at[pl.ds(base, TPW)])


def _sc_sparse_part(logits, label_ids):
    mesh = plsc.VectorSubcoreMesh(core_axis_name="c", subcore_axis_name="s",
                                  num_cores=NC, num_subcores=NS)
    k = pl.kernel(
        _sc_body,
        out_type=(jax.ShapeDtypeStruct((N,), jnp.float32),
                  jax.ShapeDtypeStruct((N,), jnp.float32)),
        mesh=mesh,
        scratch_types=[
            pltpu.VMEM((TPW + 16,), jnp.int32),
            pltpu.VMEM((HALF, 128), jnp.float32),
            pltpu.VMEM((TPW,), jnp.float32),
            pltpu.VMEM((TPW,), jnp.float32),
        ],
        compiler_params=pltpu.CompilerParams(needs_layout_passes=False),
    )
    return k(label_ids, logits)


def _tc_lse_body(x_ref, lab_ref, lse_ref, own_ref):
    i = pl.program_id(0)
    x = x_ref[0]
    r0 = lab_ref[...][:, None]
    bnd0 = (r0 < 16) & (r0 % 2 == 1)
    iota_l = jax.lax.broadcasted_iota(jnp.int32, (ROWS, L), 1)
    g0 = jnp.sum(jnp.where(iota_l == r0, x, 0.0), axis=1, keepdims=True)
    m = jnp.max(x, axis=1, keepdims=True)
    lse = m + jnp.log(jnp.sum(jnp.exp(x - m), axis=1, keepdims=True))
    lse_ref[...] = lse[:, 0]

    @pl.when(i == 0)
    def _():
        own_ref[0, 0, 0] = 0.0
    own_ref[0, 0, 0] += jnp.sum(jnp.where(bnd0, 0.0, g0))


def _tc_combine_body(lse_ref, dot_ref, wsum_ref, own_ref, out_ref):
    tot = jnp.sum(wsum_ref[...] * lse_ref[...] - dot_ref[...])
    out_ref[0, 0, 0] = (tot - own_ref[0, 0, 0]) / N


@jax.jit
def kernel(logits, label_ids):
    dot, wsum = _sc_sparse_part(logits, label_ids)
    lse, own = pl.pallas_call(
        _tc_lse_body,
        grid=(NBLK,),
        in_specs=[
            pl.BlockSpec((1, ROWS, L), lambda i: (i, 0, 0)),
            pl.BlockSpec((ROWS,), lambda i: (i,)),
        ],
        out_specs=[
            pl.BlockSpec((ROWS,), lambda i: (i,)),
            pl.BlockSpec((1, 1, 1), lambda i: (0, 0, 0),
                         memory_space=pltpu.SMEM),
        ],
        out_shape=[
            jax.ShapeDtypeStruct((N,), jnp.float32),
            jax.ShapeDtypeStruct((1, 1, 1), jnp.float32),
        ],
    )(logits.reshape(NBLK, ROWS, L), label_ids)
    combined = pl.pallas_call(
        _tc_combine_body,
        grid=(1,),
        in_specs=[
            pl.BlockSpec((N,), lambda i: (0,)),
            pl.BlockSpec((N,), lambda i: (0,)),
            pl.BlockSpec((N,), lambda i: (0,)),
            pl.BlockSpec(memory_space=pltpu.SMEM),
        ],
        out_specs=pl.BlockSpec((1, 1, 1), lambda i: (0, 0, 0),
                               memory_space=pltpu.SMEM),
        out_shape=jax.ShapeDtypeStruct((1, 1, 1), jnp.float32),
    )(lse, dot, wsum, own)
    return combined[0, 0, 0]
